# baseline (speedup 1.0000x reference)
import functools

import jax
import jax.numpy as jnp
from jax import lax
from jax.experimental import pallas as pl
from jax.experimental.pallas import tpu as pltpu
from jax.experimental.pallas import tpu_sc as plsc

NUM_NODES = 1000000
EMBED_DIM = 32
BATCH = 16384

NC = 2
NS = 16
NW = NC * NS
RANGE = 32768
PIECE = 1024
PPW = RANGE // PIECE
TAIL_BASE = 999424
TRASH = BATCH
CAP = 18432
ECHUNKS = CAP // 128
B_PER_W = BATCH // NW
RING = 4

_i32 = jnp.int32


@functools.partial(
    pl.kernel,
    mesh=plsc.VectorSubcoreMesh(core_axis_name="c", subcore_axis_name="s"),
    out_type=(
        jax.ShapeDtypeStruct((NW, CAP, 128), jnp.float32),
        jax.ShapeDtypeStruct((NW, ECHUNKS, 128), _i32),
        jax.ShapeDtypeStruct((NW, 1, 128), _i32),
    ),
    scratch_types=[
        pltpu.VMEM((BATCH,), _i32),
        pltpu.VMEM((BATCH + 16,), _i32),
        pltpu.VMEM((BATCH + 16,), _i32),
        pltpu.VMEM((32, 2 * PIECE), jnp.float32),
        pltpu.VMEM((RING, 16, 128), jnp.float32),
        pltpu.VMEM((8, 128), _i32),
        pltpu.VMEM((1, 128), _i32),
        pltpu.SemaphoreType.DMA,
        pltpu.SemaphoreType.DMA,
    ],
    compiler_params=pltpu.CompilerParams(needs_layout_passes=False),
)
def _sc_extract(idx_hbm, tab_t_hbm, tail_hbm, rows_hbm, ent_hbm, cnt_hbm,
                ids_v, bucket_v, plist_v, sbuf_v, stage_v, pend_v, cstg_v,
                sem_s, sem_w):
    wid = lax.axis_index("s") * NC + lax.axis_index("c")
    node_base = wid * RANGE
    n_pieces = jnp.where(wid < 30, PPW, jnp.where(wid == 30, 17, 0))
    iota = lax.iota(_i32, 16)
    trash_vec = jnp.full((16,), TRASH << 15, _i32)

    def fire_piece(p):
        half = lax.bitwise_and(p, 1) * PIECE
        is_tail = lax.bitwise_and(wid == 30, p == 16)

        @pl.when(jnp.logical_not(is_tail))
        def _():
            src = node_base + p * PIECE
            pltpu.async_copy(
                tab_t_hbm.at[:, pl.ds(pl.multiple_of(src, PIECE), PIECE)],
                sbuf_v.at[:, pl.ds(pl.multiple_of(half, PIECE), PIECE)],
                sem_s)

        @pl.when(is_tail)
        def _():
            pltpu.async_copy(
                tab_t_hbm.at[:, pl.ds(pl.multiple_of(TAIL_BASE, 128), 512)],
                sbuf_v.at[:, pl.ds(0, 512)], sem_s)
            pltpu.async_copy(tail_hbm, sbuf_v.at[:, pl.ds(512, 128)], sem_s)

    def wait_piece(p):
        is_tail = lax.bitwise_and(wid == 30, p == 16)

        @pl.when(jnp.logical_not(is_tail))
        def _():
            pltpu.make_async_copy(
                tab_t_hbm.at[:, pl.ds(0, PIECE)],
                sbuf_v.at[:, pl.ds(0, PIECE)], sem_s).wait()

        @pl.when(is_tail)
        def _():
            pltpu.make_async_copy(
                tab_t_hbm.at[:, pl.ds(0, 512)],
                sbuf_v.at[:, pl.ds(0, 512)], sem_s).wait()
            pltpu.make_async_copy(
                tab_t_hbm.at[:, pl.ds(0, 128)],
                sbuf_v.at[:, pl.ds(0, 128)], sem_s).wait()

    @pl.when(n_pieces > 0)
    def _():
        fire_piece(jnp.int32(0))

    @pl.when(n_pieces > 1)
    def _():
        fire_piece(jnp.int32(1))

    pltpu.sync_copy(idx_hbm, ids_v)

    def scan1(u, cnt):
        for k in range(2):
            v = u * 2 + k
            n = ids_v[pl.ds(v * 16, 16)]
            mask = lax.shift_right_logical(n, 15) == wid
            b = v * 16 + iota
            packed = lax.bitwise_or(lax.bitwise_and(n, 32767),
                                    lax.shift_left(b, 15))
            mi = mask.astype(_i32)
            dst = cnt + plsc.cumsum(mi) - mi
            plsc.store_scatter(bucket_v, [dst], packed, mask=mask)
            cnt = cnt + jnp.sum(mi)
        return cnt

    bcnt = lax.fori_loop(0, BATCH // 32, scan1, jnp.int32(0))
    n_bvecs = lax.shift_right_logical(bcnt + 15, 4)

    def piece(p, carry):
        rowoff, fp, frow = carry

        @pl.when(jnp.logical_and(p >= 1, p + 1 < n_pieces))
        def _():
            fire_piece(p + 1)

        wait_piece(p)
        off = lax.bitwise_and(p, 1) * PIECE

        def scan2(v, pcnt):
            e = bucket_v[pl.ds(v * 16, 16)]
            nl = lax.bitwise_and(e, 32767)
            mask = lax.bitwise_and(
                lax.shift_right_logical(nl, 10) == p,
                v * 16 + iota < bcnt)
            mi = mask.astype(_i32)
            dst = pcnt + plsc.cumsum(mi) - mi
            plsc.store_scatter(plist_v, [dst], e, mask=mask)
            return pcnt + jnp.sum(mi)

        pcnt = lax.fori_loop(0, n_bvecs, scan2, jnp.int32(0))
        n_groups = lax.shift_right_logical(pcnt + 15, 4)

        def group(g, carry2):
            fp2, frow2 = carry2
            par = lax.bitwise_and(g, RING - 1)

            @pl.when(g >= RING)
            def _():
                pltpu.make_async_copy(
                    rows_hbm.at[0, pl.ds(0, 16)], stage_v.at[0], sem_w).wait()

            e = plist_v[pl.ds(g * 16, 16)]
            valid = g * 16 + iota < pcnt
            e = jnp.where(valid, e, trash_vec)
            l = lax.bitwise_and(e, 1023) + off
            for d in range(EMBED_DIM):
                dsplat = jnp.full((16,), d, _i32)
                vals = plsc.load_gather(sbuf_v, [dsplat, l])
                plsc.store_scatter(stage_v.at[par], [iota, dsplat], vals)
            dst0 = rowoff + g * 16
            pltpu.async_copy(
                stage_v.at[par],
                rows_hbm.at[wid, pl.ds(pl.multiple_of(dst0, 8), 16)], sem_w)
            pend_v[lax.shift_right_logical(fp2, 7),
                   pl.ds(lax.bitwise_and(fp2, 127), 16)] = e
            fp2 = fp2 + 16

            @pl.when(fp2 == 1024)
            def _():
                pltpu.sync_copy(
                    pend_v,
                    ent_hbm.at[wid, pl.ds(pl.multiple_of(frow2, 8), 8)])

            frow2 = jnp.where(fp2 == 1024, frow2 + 8, frow2)
            fp2 = jnp.where(fp2 == 1024, 0, fp2)
            return fp2, frow2

        fp, frow = lax.fori_loop(0, n_groups, group, (fp, frow))

        def drain(i, c):
            pltpu.make_async_copy(
                rows_hbm.at[0, pl.ds(0, 16)], stage_v.at[0], sem_w).wait()
            return c

        lax.fori_loop(0, jnp.minimum(n_groups, RING), drain, jnp.int32(0))
        return rowoff + n_groups * 16, fp, frow

    rowoff, fp, frow = lax.fori_loop(
        0, n_pieces, piece,
        (jnp.int32(0), jnp.int32(0), jnp.int32(0)))

    def pad(i, c):
        fpos = fp + i * 16
        pend_v[lax.shift_right_logical(fpos, 7),
               pl.ds(lax.bitwise_and(fpos, 127), 16)] = trash_vec
        return c

    lax.fori_loop(0, lax.shift_right_logical(1024 - fp, 4), pad, jnp.int32(0))
    pltpu.sync_copy(pend_v,
                    ent_hbm.at[wid, pl.ds(pl.multiple_of(frow, 8), 8)])
    cstg_v[0, pl.ds(0, 16)] = jnp.full((16,), 1, _i32) * rowoff
    pltpu.sync_copy(cstg_v, cnt_hbm.at[wid])


@functools.partial(
    pl.kernel,
    mesh=plsc.VectorSubcoreMesh(core_axis_name="c", subcore_axis_name="s"),
    out_type=jax.ShapeDtypeStruct((BATCH, 128), jnp.float32),
    scratch_types=[
        pltpu.VMEM((2, 8, 128), _i32),
        pltpu.VMEM((8, 128), _i32),
        pltpu.VMEM((NW, 128), _i32),
        pltpu.VMEM((4, 128), _i32),
        pltpu.VMEM((B_PER_W, 128), jnp.float32),
        pltpu.SemaphoreType.DMA,
    ],
    compiler_params=pltpu.CompilerParams(needs_layout_passes=False),
)
def _sc_permute(ent_hbm, cnt_hbm, rows2_hbm, out_hbm, echk_v, xchk_v, rcnt_v,
                inv_v, rows_v, sem):
    wid = lax.axis_index("s") * NC + lax.axis_index("c")
    base = wid * B_PER_W
    iota = lax.iota(_i32, 16)

    pltpu.sync_copy(cnt_hbm, rcnt_v)
    pltpu.async_copy(ent_hbm.at[0, pl.ds(0, 8)], echk_v.at[0], sem)

    def region(r, carry):
        cnt = rcnt_v[r, pl.ds(0, 16)][0]
        n_chunks = lax.shift_right_logical(cnt + 1023, 10)
        par = lax.bitwise_and(r, 1)
        pltpu.make_async_copy(
            ent_hbm.at[0, pl.ds(0, 8)], echk_v.at[0], sem).wait()

        @pl.when(r + 1 < NW)
        def _():
            pltpu.async_copy(ent_hbm.at[r + 1, pl.ds(0, 8)],
                             echk_v.at[lax.bitwise_and(r + 1, 1)], sem)

        def scan_chunk(c, cref):
            def vec(v, carry3):
                e = cref[lax.shift_right_logical(v, 3),
                         pl.ds(lax.bitwise_and(v, 7) * 16, 16)]
                b = lax.shift_right_logical(e, 15) - base
                mask = lax.bitwise_and(b >= 0, b < B_PER_W)
                bs = jnp.where(mask, b, 0)
                slot = r * CAP + c * 1024 + v * 16 + iota
                plsc.store_scatter(
                    inv_v, [lax.shift_right_logical(bs, 7),
                            lax.bitwise_and(bs, 127)], slot, mask=mask)
                return carry3

            nvec = jnp.minimum(
                jnp.int32(64),
                lax.shift_right_logical(cnt - c * 1024 + 15, 4))
            lax.fori_loop(0, nvec, vec, jnp.int32(0))

        @pl.when(n_chunks > 0)
        def _():
            scan_chunk(jnp.int32(0), echk_v.at[par])

        def extra(c, carry2):
            pltpu.sync_copy(
                ent_hbm.at[r, pl.ds(pl.multiple_of(c * 8, 8), 8)], xchk_v)
            scan_chunk(c, xchk_v)
            return carry2

        lax.fori_loop(1, n_chunks, extra, jnp.int32(0))
        return carry

    lax.fori_loop(0, NW, region, jnp.int32(0))

    copies = [
        pltpu.async_copy(rows2_hbm.at[inv_v.at[j]],
                         rows_v.at[pl.ds(j * 128, 128)], sem)
        for j in range(4)
    ]
    for c in copies:
        c.wait()
    pltpu.sync_copy(rows_v, out_hbm.at[pl.ds(base, B_PER_W)])


def kernel(node_ids, embedding_weight):
    idx = node_ids.astype(_i32)
    tail = jnp.pad(embedding_weight[TAIL_BASE + 512:].T, ((0, 0), (0, 64)))
    rows, ents, cnts = _sc_extract(idx, embedding_weight.T, tail)
    out_pad = _sc_permute(ents, cnts.reshape(NW, 128), rows.reshape(NW * CAP, 128))
    return out_pad[:, :EMBED_DIM]


# speedup vs baseline: 1.0095x; 1.0095x over previous
import functools

import jax
import jax.numpy as jnp
from jax import lax
from jax.experimental import pallas as pl
from jax.experimental.pallas import tpu as pltpu
from jax.experimental.pallas import tpu_sc as plsc

NUM_NODES = 1000000
EMBED_DIM = 32
BATCH = 16384

NC = 2
NS = 16
NW = NC * NS
RANGE = 32768
PIECE = 1024
PPW = RANGE // PIECE
TAIL_BASE = 999424
TRASH = BATCH
CAP = 18432
ECHUNKS = CAP // 128
B_PER_W = BATCH // NW
RING = 4

_i32 = jnp.int32


@functools.partial(
    pl.kernel,
    mesh=plsc.VectorSubcoreMesh(core_axis_name="c", subcore_axis_name="s"),
    out_type=(
        jax.ShapeDtypeStruct((NW, CAP, 128), jnp.float32),
        jax.ShapeDtypeStruct((NW, ECHUNKS, 128), _i32),
        jax.ShapeDtypeStruct((NW, 1, 128), _i32),
    ),
    scratch_types=[
        pltpu.VMEM((BATCH,), _i32),
        pltpu.VMEM((BATCH + 16,), _i32),
        pltpu.VMEM((BATCH + 16,), _i32),
        pltpu.VMEM((32, 2 * PIECE), jnp.float32),
        pltpu.VMEM((RING, 16, 128), jnp.float32),
        pltpu.VMEM((8, 128), _i32),
        pltpu.VMEM((1, 128), _i32),
        pltpu.SemaphoreType.DMA,
        pltpu.SemaphoreType.DMA,
    ],
    compiler_params=pltpu.CompilerParams(needs_layout_passes=False),
)
def _sc_extract(idx_hbm, tab_t_hbm, tail_hbm, rows_hbm, ent_hbm, cnt_hbm,
                ids_v, bucket_v, plist_v, sbuf_v, stage_v, pend_v, cstg_v,
                sem_s, sem_w):
    wid = lax.axis_index("s") * NC + lax.axis_index("c")
    node_base = wid * RANGE
    n_pieces = jnp.where(wid < 30, PPW, jnp.where(wid == 30, 17, 0))
    iota = lax.iota(_i32, 16)
    trash_vec = jnp.full((16,), TRASH << 15, _i32)

    def fire_piece(p):
        half = lax.bitwise_and(p, 1) * PIECE
        is_tail = lax.bitwise_and(wid == 30, p == 16)

        @pl.when(jnp.logical_not(is_tail))
        def _():
            src = node_base + p * PIECE
            pltpu.async_copy(
                tab_t_hbm.at[:, pl.ds(pl.multiple_of(src, PIECE), PIECE)],
                sbuf_v.at[:, pl.ds(pl.multiple_of(half, PIECE), PIECE)],
                sem_s)

        @pl.when(is_tail)
        def _():
            pltpu.async_copy(
                tab_t_hbm.at[:, pl.ds(pl.multiple_of(TAIL_BASE, 128), 512)],
                sbuf_v.at[:, pl.ds(0, 512)], sem_s)
            pltpu.async_copy(tail_hbm, sbuf_v.at[:, pl.ds(512, 128)], sem_s)

    def wait_piece(p):
        is_tail = lax.bitwise_and(wid == 30, p == 16)

        @pl.when(jnp.logical_not(is_tail))
        def _():
            pltpu.make_async_copy(
                tab_t_hbm.at[:, pl.ds(0, PIECE)],
                sbuf_v.at[:, pl.ds(0, PIECE)], sem_s).wait()

        @pl.when(is_tail)
        def _():
            pltpu.make_async_copy(
                tab_t_hbm.at[:, pl.ds(0, 512)],
                sbuf_v.at[:, pl.ds(0, 512)], sem_s).wait()
            pltpu.make_async_copy(
                tab_t_hbm.at[:, pl.ds(0, 128)],
                sbuf_v.at[:, pl.ds(0, 128)], sem_s).wait()

    @pl.when(n_pieces > 0)
    def _():
        fire_piece(jnp.int32(0))

    @pl.when(n_pieces > 1)
    def _():
        fire_piece(jnp.int32(1))

    pltpu.sync_copy(idx_hbm, ids_v)

    def scan1(u, cnt):
        for k in range(2):
            v = u * 2 + k
            n = ids_v[pl.ds(v * 16, 16)]
            mask = lax.shift_right_logical(n, 15) == wid
            b = v * 16 + iota
            packed = lax.bitwise_or(lax.bitwise_and(n, 32767),
                                    lax.shift_left(b, 15))
            mi = mask.astype(_i32)
            dst = cnt + plsc.cumsum(mi) - mi
            plsc.store_scatter(bucket_v, [dst], packed, mask=mask)
            cnt = cnt + jnp.sum(mi)
        return cnt

    bcnt = lax.fori_loop(0, BATCH // 32, scan1, jnp.int32(0))
    n_bvecs = lax.shift_right_logical(bcnt + 15, 4)

    def piece(p, carry):
        rowoff, fp, frow = carry

        @pl.when(jnp.logical_and(p >= 1, p + 1 < n_pieces))
        def _():
            fire_piece(p + 1)

        wait_piece(p)
        off = lax.bitwise_and(p, 1) * PIECE

        def scan2(v, pcnt):
            e = bucket_v[pl.ds(v * 16, 16)]
            nl = lax.bitwise_and(e, 32767)
            mask = lax.bitwise_and(
                lax.shift_right_logical(nl, 10) == p,
                v * 16 + iota < bcnt)
            mi = mask.astype(_i32)
            dst = pcnt + plsc.cumsum(mi) - mi
            plsc.store_scatter(plist_v, [dst], e, mask=mask)
            return pcnt + jnp.sum(mi)

        pcnt = lax.fori_loop(0, n_bvecs, scan2, jnp.int32(0))
        n_groups = lax.shift_right_logical(pcnt + 15, 4)

        def group(g, carry2):
            fp2, frow2 = carry2
            gg = lax.shift_right_logical(rowoff, 4) + g
            par = lax.bitwise_and(gg, RING - 1)

            @pl.when(gg >= RING)
            def _():
                pltpu.make_async_copy(
                    rows_hbm.at[0, pl.ds(0, 16)], stage_v.at[0], sem_w).wait()

            e = plist_v[pl.ds(g * 16, 16)]
            valid = g * 16 + iota < pcnt
            e = jnp.where(valid, e, trash_vec)
            l = lax.bitwise_and(e, 1023) + off
            for d in range(EMBED_DIM):
                dsplat = jnp.full((16,), d, _i32)
                vals = plsc.load_gather(sbuf_v, [dsplat, l])
                plsc.store_scatter(stage_v.at[par], [iota, dsplat], vals)
            dst0 = rowoff + g * 16
            pltpu.async_copy(
                stage_v.at[par],
                rows_hbm.at[wid, pl.ds(pl.multiple_of(dst0, 8), 16)], sem_w)
            pend_v[lax.shift_right_logical(fp2, 7),
                   pl.ds(lax.bitwise_and(fp2, 127), 16)] = e
            fp2 = fp2 + 16

            @pl.when(fp2 == 1024)
            def _():
                pltpu.sync_copy(
                    pend_v,
                    ent_hbm.at[wid, pl.ds(pl.multiple_of(frow2, 8), 8)])

            frow2 = jnp.where(fp2 == 1024, frow2 + 8, frow2)
            fp2 = jnp.where(fp2 == 1024, 0, fp2)
            return fp2, frow2

        fp, frow = lax.fori_loop(0, n_groups, group, (fp, frow))
        return rowoff + n_groups * 16, fp, frow

    rowoff, fp, frow = lax.fori_loop(
        0, n_pieces, piece,
        (jnp.int32(0), jnp.int32(0), jnp.int32(0)))

    def drain(i, c):
        pltpu.make_async_copy(
            rows_hbm.at[0, pl.ds(0, 16)], stage_v.at[0], sem_w).wait()
        return c

    lax.fori_loop(0, jnp.minimum(lax.shift_right_logical(rowoff, 4),
                                 jnp.int32(RING)), drain, jnp.int32(0))

    def pad(i, c):
        fpos = fp + i * 16
        pend_v[lax.shift_right_logical(fpos, 7),
               pl.ds(lax.bitwise_and(fpos, 127), 16)] = trash_vec
        return c

    lax.fori_loop(0, lax.shift_right_logical(1024 - fp, 4), pad, jnp.int32(0))
    pltpu.sync_copy(pend_v,
                    ent_hbm.at[wid, pl.ds(pl.multiple_of(frow, 8), 8)])
    cstg_v[0, pl.ds(0, 16)] = jnp.full((16,), 1, _i32) * rowoff
    pltpu.sync_copy(cstg_v, cnt_hbm.at[wid])


@functools.partial(
    pl.kernel,
    mesh=plsc.VectorSubcoreMesh(core_axis_name="c", subcore_axis_name="s"),
    out_type=jax.ShapeDtypeStruct((BATCH, 128), jnp.float32),
    scratch_types=[
        pltpu.VMEM((2, 8, 128), _i32),
        pltpu.VMEM((8, 128), _i32),
        pltpu.VMEM((NW, 128), _i32),
        pltpu.VMEM((4, 128), _i32),
        pltpu.VMEM((B_PER_W, 128), jnp.float32),
        pltpu.SemaphoreType.DMA,
    ],
    compiler_params=pltpu.CompilerParams(needs_layout_passes=False),
)
def _sc_permute(ent_hbm, cnt_hbm, rows2_hbm, out_hbm, echk_v, xchk_v, rcnt_v,
                inv_v, rows_v, sem):
    wid = lax.axis_index("s") * NC + lax.axis_index("c")
    base = wid * B_PER_W
    iota = lax.iota(_i32, 16)

    pltpu.sync_copy(cnt_hbm, rcnt_v)
    pltpu.async_copy(ent_hbm.at[0, pl.ds(0, 8)], echk_v.at[0], sem)

    def region(r, carry):
        cnt = rcnt_v[r, pl.ds(0, 16)][0]
        n_chunks = lax.shift_right_logical(cnt + 1023, 10)
        par = lax.bitwise_and(r, 1)
        pltpu.make_async_copy(
            ent_hbm.at[0, pl.ds(0, 8)], echk_v.at[0], sem).wait()

        @pl.when(r + 1 < NW)
        def _():
            pltpu.async_copy(ent_hbm.at[r + 1, pl.ds(0, 8)],
                             echk_v.at[lax.bitwise_and(r + 1, 1)], sem)

        def scan_chunk(c, cref):
            def vec(v, carry3):
                e = cref[lax.shift_right_logical(v, 3),
                         pl.ds(lax.bitwise_and(v, 7) * 16, 16)]
                b = lax.shift_right_logical(e, 15) - base
                mask = lax.bitwise_and(b >= 0, b < B_PER_W)
                bs = jnp.where(mask, b, 0)
                slot = r * CAP + c * 1024 + v * 16 + iota
                plsc.store_scatter(
                    inv_v, [lax.shift_right_logical(bs, 7),
                            lax.bitwise_and(bs, 127)], slot, mask=mask)
                return carry3

            nvec = jnp.minimum(
                jnp.int32(64),
                lax.shift_right_logical(cnt - c * 1024 + 15, 4))
            lax.fori_loop(0, nvec, vec, jnp.int32(0))

        @pl.when(n_chunks > 0)
        def _():
            scan_chunk(jnp.int32(0), echk_v.at[par])

        def extra(c, carry2):
            pltpu.sync_copy(
                ent_hbm.at[r, pl.ds(pl.multiple_of(c * 8, 8), 8)], xchk_v)
            scan_chunk(c, xchk_v)
            return carry2

        lax.fori_loop(1, n_chunks, extra, jnp.int32(0))
        return carry

    lax.fori_loop(0, NW, region, jnp.int32(0))

    copies = [
        pltpu.async_copy(rows2_hbm.at[inv_v.at[j]],
                         rows_v.at[pl.ds(j * 128, 128)], sem)
        for j in range(4)
    ]
    for c in copies:
        c.wait()
    pltpu.sync_copy(rows_v, out_hbm.at[pl.ds(base, B_PER_W)])


def kernel(node_ids, embedding_weight):
    idx = node_ids.astype(_i32)
    tail = jnp.pad(embedding_weight[TAIL_BASE + 512:].T, ((0, 0), (0, 64)))
    rows, ents, cnts = _sc_extract(idx, embedding_weight.T, tail)
    out_pad = _sc_permute(ents, cnts.reshape(NW, 128), rows.reshape(NW * CAP, 128))
    return out_pad[:, :EMBED_DIM]
